# baseline (speedup 1.0000x reference)
import functools

import jax
import jax.numpy as jnp
import numpy as np
from jax import lax
from jax.experimental import pallas as pl
from jax.experimental.pallas import tpu as pltpu


def _round_up(x, m):
    return (x + m - 1) // m * m


def _out_dim(x):
    return (x - 1) // 2 + 1


def _sel_matrices(h_in, w_in, ho, wo):
    s = np.zeros((9, h_in * w_in, ho * wo), np.float32)
    for ky in range(3):
        for kx in range(3):
            t = 3 * ky + kx
            for a in range(ho):
                for b in range(wo):
                    i, j = 2 * a + ky - 1, 2 * b + kx - 1
                    if 0 <= i < h_in and 0 <= j < w_in:
                        s[t, i * w_in + j, a * wo + b] = 1.0
    return jnp.asarray(s, jnp.bfloat16)


def _invariant_spec(block_shape, index_map, single_buffer):
    if single_buffer:
        try:
            return pl.BlockSpec(block_shape, index_map,
                                pipeline_mode=pl.Buffered(1))
        except TypeError:
            pass
    return pl.BlockSpec(block_shape, index_map)


def _fused_p6p7_kernel(xq_ref, w6_ref, b6_ref, w7_ref, b7_ref, st_ref,
                       p6_ref, p7_ref, *, ho, wo, wb, cin, cout):
    mm = ho * wo
    x = xq_ref[0]

    taps = []
    for r in range(2):
        for s in range(2):
            p = 2 * r + s
            for db in range(2 - s):
                win = x[:, p * wb + db:p * wb + db + wo, :]
                for da in range(2 - r):
                    t = (2 * da + r) * 3 + (2 * db + s)
                    taps.append((t, win[da:da + ho].reshape(mm, cin)))
    taps.sort(key=lambda kv: kv[0])
    a_all = jnp.concatenate([a for _, a in taps], axis=1)
    acc1t = lax.dot_general(w6_ref[...], a_all,
                            (((0,), (1,)), ((), ())),
                            preferred_element_type=jnp.float32)
    p6v = acc1t + b6_ref[...]
    p6_ref[0] = p6v

    x1t = jnp.maximum(p6v, 0.0).astype(jnp.bfloat16)
    gathers = []
    for t in range(9):
        g = jnp.dot(x1t, st_ref[t], preferred_element_type=jnp.float32)
        gathers.append(g.astype(jnp.bfloat16))
    a2 = jnp.concatenate(gathers, axis=0)
    out2t = lax.dot_general(w7_ref[...], a2,
                            (((0,), (0,)), ((), ())),
                            preferred_element_type=jnp.float32)
    p7_ref[0] = out2t + b7_ref[...]


def kernel(p6_w, p6_b, p7_w, p7_b, c5):
    n, cin, h, w = c5.shape
    cout = p6_w.shape[0]
    ho, wo = _out_dim(h), _out_dim(w)
    ho2, wo2 = _out_dim(ho), _out_dim(wo)
    mm, mm2 = ho * wo, ho2 * wo2
    wb = _round_up(wo + 1, 4)

    xb = c5.astype(jnp.bfloat16)
    xp = jnp.pad(xb, ((0, 0), (0, 0),
                      (1, 2 * (ho + 1) - h - 1), (1, 2 * wb - w - 1)))
    xp = xp.reshape(n, cin, ho + 1, 2, wb, 2)
    xq = jnp.transpose(xp, (0, 2, 3, 5, 4, 1)).reshape(n, ho + 1, 4 * wb, cin)

    w6c = jnp.transpose(p6_w, (2, 3, 1, 0)).reshape(9 * cin, cout)
    w6c = w6c.astype(jnp.bfloat16)
    w7c = jnp.transpose(p7_w, (2, 3, 1, 0)).reshape(9 * cout, cout)
    w7c = w7c.astype(jnp.bfloat16)
    b6c = p6_b.reshape(cout, 1).astype(jnp.float32)
    b7c = p7_b.reshape(cout, 1).astype(jnp.float32)
    stm = _sel_matrices(ho, wo, ho2, wo2)

    kfn = functools.partial(_fused_p6p7_kernel, ho=ho, wo=wo, wb=wb,
                            cin=cin, cout=cout)
    flops = 2 * n * mm * 9 * cin * cout + 4 * n * mm * 9 * cout * mm2
    bytes_accessed = int(xq.size * 2 + w6c.size * 2 + w7c.size * 2
                         + n * (mm + mm2) * cout * 4)
    w6_single = w6c.size * 2 >= 512 * 1024

    p6f, p7f = pl.pallas_call(
        kfn,
        out_shape=[jax.ShapeDtypeStruct((n, cout, mm), jnp.float32),
                   jax.ShapeDtypeStruct((n, cout, mm2), jnp.float32)],
        grid=(n,),
        in_specs=[
            pl.BlockSpec((1, ho + 1, 4 * wb, cin), lambda i: (i, 0, 0, 0)),
            _invariant_spec((9 * cin, cout), lambda i: (0, 0), w6_single),
            _invariant_spec((cout, 1), lambda i: (0, 0), False),
            _invariant_spec((9 * cout, cout), lambda i: (0, 0), False),
            _invariant_spec((cout, 1), lambda i: (0, 0), False),
            _invariant_spec((9, mm, mm2), lambda i: (0, 0, 0), False),
        ],
        out_specs=[pl.BlockSpec((1, cout, mm), lambda i: (i, 0, 0)),
                   pl.BlockSpec((1, cout, mm2), lambda i: (i, 0, 0))],
        compiler_params=pltpu.CompilerParams(
            dimension_semantics=("parallel",),
            vmem_limit_bytes=64 * 1024 * 1024),
        cost_estimate=pl.CostEstimate(flops=flops, transcendentals=0,
                                      bytes_accessed=bytes_accessed),
    )(xq, w6c, b6c, w7c, b7c, stm)

    p6 = p6f.reshape(n, cout, ho, wo)
    p7 = p7f.reshape(n, cout, ho2, wo2)
    return [p6, p7]


# speedup vs baseline: 1.6426x; 1.6426x over previous
import functools

import jax
import jax.numpy as jnp
import numpy as np
from jax import lax
from jax.experimental import pallas as pl
from jax.experimental.pallas import tpu as pltpu


def _out_dim(x):
    return (x - 1) // 2 + 1


def _sel_matrices(h_in, w_in, ho, wo):
    s = np.zeros((9, ho * wo, h_in * w_in), np.float32)
    for ky in range(3):
        for kx in range(3):
            t = 3 * ky + kx
            for a in range(ho):
                for b in range(wo):
                    i, j = 2 * a + ky - 1, 2 * b + kx - 1
                    if 0 <= i < h_in and 0 <= j < w_in:
                        s[t, a * wo + b, i * w_in + j] = 1.0
    return jnp.asarray(s, jnp.bfloat16)


def _invariant_spec(block_shape, index_map, single_buffer):
    if single_buffer:
        try:
            return pl.BlockSpec(block_shape, index_map,
                                pipeline_mode=pl.Buffered(1))
        except TypeError:
            pass
    return pl.BlockSpec(block_shape, index_map)


def _deinterleave_rows(x2m):
    xi = pltpu.bitcast(x2m, jnp.int32)
    lo = lax.bitcast_convert_type(xi.astype(jnp.int16), jnp.bfloat16)
    hi = lax.bitcast_convert_type(
        lax.shift_right_logical(xi, jnp.int32(16)).astype(jnp.int16),
        jnp.bfloat16)
    return lo, hi


def _fused_p6p7_kernel(x_ref, w6_ref, b6_ref, w7_ref, b7_ref, st_ref,
                       p6_ref, p7_ref, se_ref, so_ref, sm_ref, *,
                       h, w, ho, wo, cin, cout):
    mm = ho * wo
    xb = x_ref[0].astype(jnp.bfloat16)
    xt = jnp.transpose(xb)
    xev, xod = _deinterleave_rows(xt)
    xev = xev.reshape(h, wo, cin)
    xod = xod.reshape(h, wo, cin)
    zrow = jnp.zeros((h, 1, cin), jnp.bfloat16)
    xodm = jnp.concatenate([zrow, xod[:, :wo - 1, :]], axis=1)
    se_ref[...] = xev.reshape(ho, 2, wo, cin)
    so_ref[...] = xod.reshape(ho, 2, wo, cin)
    sm_ref[...] = xodm.reshape(ho, 2, wo, cin)
    wplane = {0: sm_ref, 1: se_ref, 2: so_ref}

    zplane = jnp.zeros((1, wo, cin), jnp.bfloat16)
    acc = None
    for ky in range(3):
        for kx in range(3):
            v = wplane[kx]
            if ky == 0:
                a_t = jnp.concatenate([zplane, v[0:ho - 1, 1]], axis=0)
            elif ky == 1:
                a_t = v[:, 0]
            else:
                a_t = v[:, 1]
            a_t = a_t.reshape(mm, cin)
            d = jnp.dot(a_t, w6_ref[3 * ky + kx],
                        preferred_element_type=jnp.float32)
            acc = d if acc is None else acc + d
    p6v = acc + b6_ref[...]
    p6_ref[0] = jnp.transpose(p6v)

    x1 = jnp.maximum(p6v, 0.0).astype(jnp.bfloat16)
    gathers = []
    for t in range(9):
        g = jnp.dot(st_ref[t], x1, preferred_element_type=jnp.float32)
        gathers.append(g.astype(jnp.bfloat16))
    a2 = jnp.concatenate(gathers, axis=1)
    out2 = jnp.dot(a2, w7_ref[...],
                   preferred_element_type=jnp.float32)
    p7_ref[0] = jnp.transpose(out2 + b7_ref[...])


def kernel(p6_w, p6_b, p7_w, p7_b, c5):
    n, cin, h, w = c5.shape
    cout = p6_w.shape[0]
    ho, wo = _out_dim(h), _out_dim(w)
    ho2, wo2 = _out_dim(ho), _out_dim(wo)
    mm, mm2 = ho * wo, ho2 * wo2

    x3 = c5.reshape(n, cin, h * w)
    w6c = jnp.transpose(p6_w, (2, 3, 1, 0)).reshape(9, cin, cout)
    w6c = w6c.astype(jnp.bfloat16)
    w7c = jnp.transpose(p7_w, (2, 3, 1, 0)).reshape(9 * cout, cout)
    w7c = w7c.astype(jnp.bfloat16)
    b6c = p6_b.reshape(1, cout).astype(jnp.float32)
    b7c = p7_b.reshape(1, cout).astype(jnp.float32)
    stm = _sel_matrices(ho, wo, ho2, wo2)

    kfn = functools.partial(_fused_p6p7_kernel, h=h, w=w, ho=ho, wo=wo,
                            cin=cin, cout=cout)
    flops = 2 * n * mm * 9 * cin * cout + 4 * n * mm * 9 * cout * mm2
    bytes_accessed = int(x3.size * 4 + w6c.size * 2 + w7c.size * 2
                         + n * (mm + mm2) * cout * 4)
    w6_single = w6c.size * 2 >= 512 * 1024

    p6f, p7f = pl.pallas_call(
        kfn,
        out_shape=[jax.ShapeDtypeStruct((n, cout, mm), jnp.float32),
                   jax.ShapeDtypeStruct((n, cout, mm2), jnp.float32)],
        grid=(n,),
        in_specs=[
            pl.BlockSpec((1, cin, h * w), lambda i: (i, 0, 0)),
            _invariant_spec((9, cin, cout), lambda i: (0, 0, 0), w6_single),
            _invariant_spec((1, cout), lambda i: (0, 0), False),
            _invariant_spec((9 * cout, cout), lambda i: (0, 0), False),
            _invariant_spec((1, cout), lambda i: (0, 0), False),
            _invariant_spec((9, mm2, mm), lambda i: (0, 0, 0), False),
        ],
        out_specs=[pl.BlockSpec((1, cout, mm), lambda i: (i, 0, 0)),
                   pl.BlockSpec((1, cout, mm2), lambda i: (i, 0, 0))],
        scratch_shapes=[pltpu.VMEM((ho, 2, wo, cin), jnp.bfloat16)
                        for _ in range(3)],
        compiler_params=pltpu.CompilerParams(
            dimension_semantics=("parallel",),
            vmem_limit_bytes=64 * 1024 * 1024),
        cost_estimate=pl.CostEstimate(flops=flops, transcendentals=0,
                                      bytes_accessed=bytes_accessed),
    )(x3, w6c, b6c, w7c, b7c, stm)

    p6 = p6f.reshape(n, cout, ho, wo)
    p7 = p7f.reshape(n, cout, ho2, wo2)
    return [p6, p7]
